# Initial kernel scaffold; baseline (speedup 1.0000x reference)
import functools

import jax
import jax.numpy as jnp
from jax import lax
from jax.experimental import pallas as pl
from jax.experimental.pallas import tpu as pltpu
from jax.experimental.pallas import tpu_sc as plsc

N = 10000
E = 320000
IN_DIM = 128
H = 32
T = 4
O = 5

NC = 2
NS = 16
NW = NC * NS
NP = 10240
TPN = NP // NS
EW = E // NW
C = 2000
NCH = EW // C

_MESH = plsc.VectorSubcoreMesh(
    core_axis_name="c", subcore_axis_name="s", num_cores=NC, num_subcores=NS)


def _elu(v):
    return jnp.where(v > 0, v, jnp.expm1(v))


def _ln(v, g, b):
    mu = jnp.mean(v, axis=-1, keepdims=True)
    var = jnp.var(v, axis=-1, keepdims=True)
    return (v - mu) / jnp.sqrt(var + 1e-5) * g + b


def _zero_rows(ref, nrows, width):
    zero = jnp.zeros((16,), jnp.float32)

    def body(i, carry):
        for w0 in range(0, width, 16):
            ref[i, pl.ds(w0, 16)] = zero
        return carry

    lax.fori_loop(0, nrows, body, 0)


@functools.partial(
    pl.kernel,
    out_type=jax.ShapeDtypeStruct((NC, NP, 16), jnp.float32),
    mesh=_MESH,
    scratch_types=[
        pltpu.VMEM((C,), jnp.int32),
        pltpu.VMEM((C, 16), jnp.float32),
        pltpu.VMEM((TPN, 16), jnp.float32),
        pltpu.VMEM_SHARED((NP, 16), jnp.float32),
    ],
)
def _sc_deg(dst_hbm, out_hbm, didx, ones, stage, acc):
    cid = lax.axis_index("c")
    sid = lax.axis_index("s")
    wid = sid * NC + cid

    one = jnp.ones((16,), jnp.float32)

    def fill(i, carry):
        ones[i, :] = one
        return carry

    lax.fori_loop(0, C, fill, 0)
    _zero_rows(stage, TPN, 16)
    pltpu.sync_copy(stage, acc.at[pl.ds(sid * TPN, TPN)])
    plsc.subcore_barrier()

    def chunk(i, carry):
        b = pl.multiple_of(wid * EW + i * C, 8)
        pltpu.sync_copy(dst_hbm.at[pl.ds(b, C)], didx)
        pltpu.sync_copy(ones, acc.at[didx], add=True)
        return carry

    lax.fori_loop(0, NCH, chunk, 0)
    plsc.subcore_barrier()
    pltpu.sync_copy(acc.at[pl.ds(sid * TPN, TPN)],
                    out_hbm.at[cid, pl.ds(sid * TPN, TPN)])


@functools.partial(
    pl.kernel,
    out_type=jax.ShapeDtypeStruct((NC, NP, H), jnp.float32),
    mesh=_MESH,
    scratch_types=[
        pltpu.VMEM((C,), jnp.int32),
        pltpu.VMEM((C,), jnp.int32),
        pltpu.VMEM((C, H), jnp.float32),
        pltpu.VMEM((TPN, H), jnp.float32),
        pltpu.VMEM_SHARED((NP, H), jnp.float32),
        pltpu.SemaphoreType.DMA,
    ],
)
def _sc_gcn_edges(xws_hbm, src_hbm, dst_hbm, out_hbm,
                  sidx, didx, rows, stage, acc, sem):
    cid = lax.axis_index("c")
    sid = lax.axis_index("s")
    wid = sid * NC + cid

    _zero_rows(stage, TPN, H)
    pltpu.sync_copy(stage, acc.at[pl.ds(sid * TPN, TPN)])
    plsc.subcore_barrier()

    def chunk(i, carry):
        b = pl.multiple_of(wid * EW + i * C, 8)
        pltpu.sync_copy(src_hbm.at[pl.ds(b, C)], sidx)
        pltpu.sync_copy(dst_hbm.at[pl.ds(b, C)], didx)
        pltpu.async_copy(xws_hbm.at[sidx], rows, sem).wait()
        pltpu.sync_copy(rows, acc.at[didx], add=True)
        return carry

    lax.fori_loop(0, NCH, chunk, 0)
    plsc.subcore_barrier()
    pltpu.sync_copy(acc.at[pl.ds(sid * TPN, TPN)],
                    out_hbm.at[cid, pl.ds(sid * TPN, TPN)])


@functools.partial(
    pl.kernel,
    out_type=(jax.ShapeDtypeStruct((E, H), jnp.float32),
              jax.ShapeDtypeStruct((E, H), jnp.float32)),
    mesh=_MESH,
    scratch_types=[
        pltpu.VMEM((C,), jnp.int32),
        pltpu.VMEM((C,), jnp.int32),
        pltpu.VMEM((C, H), jnp.float32),
        pltpu.VMEM((C, H), jnp.float32),
        pltpu.SemaphoreType.DMA,
        pltpu.SemaphoreType.DMA,
    ],
)
def _sc_gather_pairs(h_hbm, src_hbm, dst_hbm, hs_hbm, hd_hbm,
                     sidx, didx, rows_s, rows_d, sem_s, sem_d):
    cid = lax.axis_index("c")
    sid = lax.axis_index("s")
    wid = sid * NC + cid

    def chunk(i, carry):
        b = pl.multiple_of(wid * EW + i * C, 8)
        pltpu.sync_copy(src_hbm.at[pl.ds(b, C)], sidx)
        pltpu.sync_copy(dst_hbm.at[pl.ds(b, C)], didx)
        cp_s = pltpu.async_copy(h_hbm.at[sidx], rows_s, sem_s)
        cp_d = pltpu.async_copy(h_hbm.at[didx], rows_d, sem_d)
        cp_s.wait()
        pltpu.sync_copy(rows_s, hs_hbm.at[pl.ds(b, C)])
        cp_d.wait()
        pltpu.sync_copy(rows_d, hd_hbm.at[pl.ds(b, C)])
        return carry

    lax.fori_loop(0, NCH, chunk, 0)


PW = 48


@functools.partial(
    pl.kernel,
    out_type=jax.ShapeDtypeStruct((NC, NP, PW), jnp.float32),
    mesh=_MESH,
    scratch_types=[
        pltpu.VMEM((C,), jnp.int32),
        pltpu.VMEM((C, PW), jnp.float32),
        pltpu.VMEM((TPN, PW), jnp.float32),
        pltpu.VMEM_SHARED((NP, PW), jnp.float32),
    ],
)
def _sc_scatter_msg(p_hbm, dst_hbm, out_hbm, didx, rows, stage, acc):
    cid = lax.axis_index("c")
    sid = lax.axis_index("s")
    wid = sid * NC + cid

    _zero_rows(stage, TPN, PW)
    pltpu.sync_copy(stage, acc.at[pl.ds(sid * TPN, TPN)])
    plsc.subcore_barrier()

    def chunk(i, carry):
        b = pl.multiple_of(wid * EW + i * C, 8)
        pltpu.sync_copy(dst_hbm.at[pl.ds(b, C)], didx)
        pltpu.sync_copy(p_hbm.at[pl.ds(b, C)], rows)
        pltpu.sync_copy(rows, acc.at[didx], add=True)
        return carry

    lax.fori_loop(0, NCH, chunk, 0)
    plsc.subcore_barrier()
    pltpu.sync_copy(acc.at[pl.ds(sid * TPN, TPN)],
                    out_hbm.at[cid, pl.ds(sid * TPN, TPN)])


RB = 2000
GRID_N = N // RB
EB = 8000
GRID_E = E // EB


def _full(shape):
    return pl.BlockSpec(shape, lambda i: tuple(0 for _ in shape))


def _rows(width):
    return pl.BlockSpec((RB, width), lambda i: (i, 0))


def _dot(a, b):
    return jnp.dot(a, b, preferred_element_type=jnp.float32)


def _tc_a_body(x, degp, ego_W1, ego_b1, ego_W2, ego_b2, g1_W,
               h_ego_o, xw1s_o, dinv_o):
    xb = x[...]
    dp = degp[...]
    h_ego_o[...] = _dot(_elu(_dot(xb, ego_W1[...]) + ego_b1[...]),
                        ego_W2[...]) + ego_b2[...]
    deg = 1.0 + dp[:, 0:1] + dp[:, 1:2]
    dinv = lax.rsqrt(deg)
    dinv_o[...] = dinv
    xw1s_o[...] = _dot(xb, g1_W[...]) * dinv


def _tc_a(x, degp, ego_W1, ego_b1, ego_W2, ego_b2, g1_W):
    return pl.pallas_call(
        _tc_a_body,
        grid=(GRID_N,),
        in_specs=[
            _rows(IN_DIM), _rows(2),
            _full((IN_DIM, H)), _full((H,)), _full((H, H)), _full((H,)),
            _full((IN_DIM, H)),
        ],
        out_specs=[_rows(H), _rows(H), _rows(1)],
        out_shape=[
            jax.ShapeDtypeStruct((N, H), jnp.float32),
            jax.ShapeDtypeStruct((N, H), jnp.float32),
            jax.ShapeDtypeStruct((N, 1), jnp.float32),
        ],
    )(x, degp, ego_W1, ego_b1, ego_W2, ego_b2, g1_W)


def _tc_b_body(a0, a1, xws, dinv, gb, lng, lnb, W2, xw2s_o):
    dv = dinv[...]
    g1out = dv * (a0[...] + a1[...] + xws[...]) + gb[...]
    h1 = _elu(_ln(g1out, lng[...], lnb[...]))
    xw2s_o[...] = _dot(h1, W2[...]) * dv


def _tc_b(a0, a1, xws, dinv, gb, lng, lnb, W2):
    return pl.pallas_call(
        _tc_b_body,
        grid=(GRID_N,),
        in_specs=[
            _rows(H), _rows(H), _rows(H), _rows(1),
            _full((H,)), _full((H,)), _full((H,)), _full((H, H)),
        ],
        out_specs=[_rows(H)],
        out_shape=[jax.ShapeDtypeStruct((N, H), jnp.float32)],
    )(a0, a1, xws, dinv, gb, lng, lnb, W2)


def _tc_c_body(a0, a1, xws, dinv, gb, lng, lnb, h_o):
    g2out = dinv[...] * (a0[...] + a1[...] + xws[...]) + gb[...]
    h_o[...] = _elu(_ln(g2out, lng[...], lnb[...]))


def _tc_c(a0, a1, xws, dinv, gb, lng, lnb):
    return pl.pallas_call(
        _tc_c_body,
        grid=(GRID_N,),
        in_specs=[
            _rows(H), _rows(H), _rows(H), _rows(1),
            _full((H,)), _full((H,)), _full((H,)),
        ],
        out_specs=[_rows(H)],
        out_shape=[jax.ShapeDtypeStruct((N, H), jnp.float32)],
    )(a0, a1, xws, dinv, gb, lng, lnb)


def _tc_d_body(hs, hd, eW1, eb1, eW2, eb2, p_o):
    hsb = hs[...]
    hdb = hd[...]
    w1 = eW1[...]
    d = hdb - hsb
    sim = jnp.exp(-jnp.sum(d * d, axis=-1, keepdims=True) / (2.0 * H))
    z = (_dot(hdb, w1[0:H]) + _dot(hsb, w1[H:2 * H])
         + sim * w1[2 * H:2 * H + 1] + eb1[...])
    u = _elu(z)
    t = jnp.sum(u * eW2[...][:, 0], axis=-1, keepdims=True) + eb2[...]
    wgt = jax.nn.sigmoid(t)
    p_o[...] = jnp.concatenate(
        [hsb * wgt, wgt, jnp.zeros((hsb.shape[0], PW - H - 1), jnp.float32)],
        axis=-1)


def _tc_d(hs, hd, eW1, eb1, eW2, eb2):
    return pl.pallas_call(
        _tc_d_body,
        grid=(GRID_E,),
        in_specs=[
            pl.BlockSpec((EB, H), lambda i: (i, 0)),
            pl.BlockSpec((EB, H), lambda i: (i, 0)),
            _full((2 * H + 1, 16)), _full((16,)), _full((16, 1)), _full((1,)),
        ],
        out_specs=[pl.BlockSpec((EB, PW), lambda i: (i, 0))],
        out_shape=[jax.ShapeDtypeStruct((E, PW), jnp.float32)],
    )(hs, hd, eW1, eb1, eW2, eb2)


def _tc_e_body(h_ego, h, ae0, ae1,
               out_W1, out_b1, out_W2, out_b2, out_W3, out_b3,
               loc_W1, loc_b1, loc_W2, loc_b2,
               mu_W1, mu_b1, mu_W2, mu_b2,
               lv_W1, lv_b1, lv_W2, lv_b2,
               yf_o, yl_o, mu_o, lv_o):
    he = h_ego[...]
    hb = h[...]
    a = ae0[...] + ae1[...]
    h_exp = a[:, 0:H] / jnp.maximum(a[:, H:H + 1], 1e-8)
    h_full = jnp.concatenate([he, hb, h_exp], axis=-1)

    def softmax(v):
        m = jnp.max(v, axis=-1, keepdims=True)
        e = jnp.exp(v - m)
        return e / jnp.sum(e, axis=-1, keepdims=True)

    o = _elu(_dot(h_full, out_W1[...]) + out_b1[...])
    o = _elu(_dot(o, out_W2[...]) + out_b2[...])
    yf_o[...] = softmax(_dot(o, out_W3[...]) + out_b3[...])
    yl_o[...] = softmax(_dot(_elu(_dot(he, loc_W1[...]) + loc_b1[...]),
                             loc_W2[...]) + loc_b2[...])
    mu_o[...] = _dot(_elu(_dot(h_full, mu_W1[...]) + mu_b1[...]),
                     mu_W2[...]) + mu_b2[...]
    lv_o[...] = jnp.clip(_dot(_elu(_dot(h_full, lv_W1[...]) + lv_b1[...]),
                              lv_W2[...]) + lv_b2[...], -5.0, 5.0)


def _tc_e(h_ego, h, ae0, ae1, *weights):
    wspecs = [_full(w.shape) for w in weights]
    return pl.pallas_call(
        _tc_e_body,
        grid=(GRID_N,),
        in_specs=[_rows(H), _rows(H), _rows(PW), _rows(PW)] + wspecs,
        out_specs=[_rows(O), _rows(O), _rows(T), _rows(T)],
        out_shape=[
            jax.ShapeDtypeStruct((N, O), jnp.float32),
            jax.ShapeDtypeStruct((N, O), jnp.float32),
            jax.ShapeDtypeStruct((N, T), jnp.float32),
            jax.ShapeDtypeStruct((N, T), jnp.float32),
        ],
    )(h_ego, h, ae0, ae1, *weights)


def kernel(x, edge_index, ego_W1, ego_b1, ego_W2, ego_b2, g1_W, g1_b,
           g2_W, g2_b, ln1_g, ln1_b, ln2_g, ln2_b, exp_W1, exp_b1,
           exp_W2, exp_b2, out_W1, out_b1, out_W2, out_b2, out_W3, out_b3,
           loc_W1, loc_b1, loc_W2, loc_b2, mu_W1, mu_b1, mu_W2, mu_b2,
           lv_W1, lv_b1, lv_W2, lv_b2):
    src = edge_index[0]
    dst = edge_index[1]

    deg_parts = _sc_deg(dst)
    degp = jnp.stack([deg_parts[0, :N, 0], deg_parts[1, :N, 0]], axis=-1)

    h_ego, xw1s, dinv = _tc_a(x, degp, ego_W1, ego_b1, ego_W2, ego_b2, g1_W)

    a1 = _sc_gcn_edges(xw1s, src, dst)
    (xw2s,) = _tc_b(a1[0, :N], a1[1, :N], xw1s, dinv,
                    g1_b, ln1_g, ln1_b, g2_W)

    a2 = _sc_gcn_edges(xw2s, src, dst)
    (h,) = _tc_c(a2[0, :N], a2[1, :N], xw2s, dinv, g2_b, ln2_g, ln2_b)

    hs, hd = _sc_gather_pairs(h, src, dst)
    (p,) = _tc_d(hs, hd, exp_W1, exp_b1, exp_W2, exp_b2)

    ae = _sc_scatter_msg(p, dst)
    yf, yl, mu, lv = _tc_e(
        h_ego, h, ae[0, :N], ae[1, :N],
        out_W1, out_b1, out_W2, out_b2, out_W3, out_b3,
        loc_W1, loc_b1, loc_W2, loc_b2,
        mu_W1, mu_b1, mu_W2, mu_b2,
        lv_W1, lv_b1, lv_W2, lv_b2)
    return (yf, yl, mu, lv)


# speedup vs baseline: 12.1709x; 12.1709x over previous
import functools

import jax
import jax.numpy as jnp
from jax import lax
from jax.experimental import pallas as pl
from jax.experimental.pallas import tpu as pltpu
from jax.experimental.pallas import tpu_sc as plsc

N = 10000
E = 320000
IN_DIM = 128
H = 32
T = 4
O = 5

NC = 2
NS = 16
NW = NC * NS
NP = 10240
TPN = NP // NS
EW = E // NW
C = 1000
NCH = EW // C

_MESH = plsc.VectorSubcoreMesh(
    core_axis_name="c", subcore_axis_name="s", num_cores=NC, num_subcores=NS)


def _elu(v):
    return jnp.where(v > 0, v, jnp.exp(jnp.minimum(v, 0.0)) - 1.0)


def _ln(v, g, b):
    mu = jnp.mean(v, axis=-1, keepdims=True)
    var = jnp.var(v, axis=-1, keepdims=True)
    return (v - mu) / jnp.sqrt(var + 1e-5) * g + b


def _zero_rows(ref, nrows, width):
    zero = jnp.zeros((16,), jnp.float32)

    def body(i, carry):
        for w0 in range(0, width, 16):
            ref[i, pl.ds(w0, 16)] = zero
        return carry

    lax.fori_loop(0, nrows, body, 0)


@functools.partial(
    pl.kernel,
    out_type=jax.ShapeDtypeStruct((NC, NP, 16), jnp.float32),
    mesh=_MESH,
    compiler_params=pltpu.CompilerParams(use_tc_tiling_on_sc=False),
    scratch_types=[
        pltpu.VMEM((C,), jnp.int32),
        pltpu.VMEM((C, 16), jnp.float32),
        pltpu.VMEM_SHARED((NP, 16), jnp.float32),
    ],
)
def _sc_deg(dst_hbm, out_hbm, didx, ones, acc):
    cid = lax.axis_index("c")
    sid = lax.axis_index("s")
    wid = sid * NC + cid

    _zero_rows(ones, TPN, 16)
    pltpu.sync_copy(ones.at[pl.ds(0, TPN)], acc.at[pl.ds(sid * TPN, TPN)])

    one = jnp.ones((16,), jnp.float32)

    def fill(i, carry):
        ones[i, :] = one
        return carry

    lax.fori_loop(0, C, fill, 0)
    plsc.subcore_barrier()

    def chunk(i, carry):
        b = pl.multiple_of(wid * EW + i * C, 8)
        pltpu.sync_copy(dst_hbm.at[pl.ds(b, C)], didx)
        pltpu.sync_copy(ones, acc.at[didx], add=True)
        return carry

    lax.fori_loop(0, NCH, chunk, 0)
    plsc.subcore_barrier()
    pltpu.sync_copy(acc.at[pl.ds(sid * TPN, TPN)],
                    out_hbm.at[cid, pl.ds(sid * TPN, TPN)])


@functools.partial(
    pl.kernel,
    out_type=jax.ShapeDtypeStruct((NC, NP, H), jnp.float32),
    mesh=_MESH,
    compiler_params=pltpu.CompilerParams(use_tc_tiling_on_sc=False),
    scratch_types=[
        pltpu.VMEM((C,), jnp.int32),
        pltpu.VMEM((C,), jnp.int32),
        pltpu.VMEM((C, H), jnp.float32),
        pltpu.VMEM_SHARED((NP, H), jnp.float32),
        pltpu.SemaphoreType.DMA,
    ],
)
def _sc_gcn_edges(xws_hbm, src_hbm, dst_hbm, out_hbm,
                  sidx, didx, rows, acc, sem):
    cid = lax.axis_index("c")
    sid = lax.axis_index("s")
    wid = sid * NC + cid

    _zero_rows(rows, TPN, H)
    pltpu.sync_copy(rows.at[pl.ds(0, TPN)], acc.at[pl.ds(sid * TPN, TPN)])
    plsc.subcore_barrier()

    def chunk(i, carry):
        b = pl.multiple_of(wid * EW + i * C, 8)
        pltpu.sync_copy(src_hbm.at[pl.ds(b, C)], sidx)
        pltpu.sync_copy(dst_hbm.at[pl.ds(b, C)], didx)
        pltpu.async_copy(xws_hbm.at[sidx], rows, sem).wait()
        pltpu.sync_copy(rows, acc.at[didx], add=True)
        return carry

    lax.fori_loop(0, NCH, chunk, 0)
    plsc.subcore_barrier()
    pltpu.sync_copy(acc.at[pl.ds(sid * TPN, TPN)],
                    out_hbm.at[cid, pl.ds(sid * TPN, TPN)])


@functools.partial(
    pl.kernel,
    out_type=(jax.ShapeDtypeStruct((E, H), jnp.float32),
              jax.ShapeDtypeStruct((E, H), jnp.float32)),
    mesh=_MESH,
    compiler_params=pltpu.CompilerParams(use_tc_tiling_on_sc=False),
    scratch_types=[
        pltpu.VMEM((C,), jnp.int32),
        pltpu.VMEM((C,), jnp.int32),
        pltpu.VMEM((C, H), jnp.float32),
        pltpu.VMEM((C, H), jnp.float32),
        pltpu.SemaphoreType.DMA,
        pltpu.SemaphoreType.DMA,
    ],
)
def _sc_gather_pairs(h_hbm, src_hbm, dst_hbm, hs_hbm, hd_hbm,
                     sidx, didx, rows_s, rows_d, sem_s, sem_d):
    cid = lax.axis_index("c")
    sid = lax.axis_index("s")
    wid = sid * NC + cid

    def chunk(i, carry):
        b = pl.multiple_of(wid * EW + i * C, 8)
        pltpu.sync_copy(src_hbm.at[pl.ds(b, C)], sidx)
        pltpu.sync_copy(dst_hbm.at[pl.ds(b, C)], didx)
        cp_s = pltpu.async_copy(h_hbm.at[sidx], rows_s, sem_s)
        cp_d = pltpu.async_copy(h_hbm.at[didx], rows_d, sem_d)
        cp_s.wait()
        pltpu.sync_copy(rows_s, hs_hbm.at[pl.ds(b, C)])
        cp_d.wait()
        pltpu.sync_copy(rows_d, hd_hbm.at[pl.ds(b, C)])
        return carry

    lax.fori_loop(0, NCH, chunk, 0)


PW = 48


@functools.partial(
    pl.kernel,
    out_type=jax.ShapeDtypeStruct((NC, NP, PW), jnp.float32),
    mesh=_MESH,
    compiler_params=pltpu.CompilerParams(use_tc_tiling_on_sc=False),
    scratch_types=[
        pltpu.VMEM((C,), jnp.int32),
        pltpu.VMEM((C, PW), jnp.float32),
        pltpu.VMEM_SHARED((NP, PW), jnp.float32),
    ],
)
def _sc_scatter_msg(p_hbm, dst_hbm, out_hbm, didx, rows, acc):
    cid = lax.axis_index("c")
    sid = lax.axis_index("s")
    wid = sid * NC + cid

    _zero_rows(rows, TPN, PW)
    pltpu.sync_copy(rows.at[pl.ds(0, TPN)], acc.at[pl.ds(sid * TPN, TPN)])
    plsc.subcore_barrier()

    def chunk(i, carry):
        b = pl.multiple_of(wid * EW + i * C, 8)
        pltpu.sync_copy(dst_hbm.at[pl.ds(b, C)], didx)
        pltpu.sync_copy(p_hbm.at[pl.ds(b, C)], rows)
        pltpu.sync_copy(rows, acc.at[didx], add=True)
        return carry

    lax.fori_loop(0, NCH, chunk, 0)
    plsc.subcore_barrier()
    pltpu.sync_copy(acc.at[pl.ds(sid * TPN, TPN)],
                    out_hbm.at[cid, pl.ds(sid * TPN, TPN)])


RB = 2000
GRID_N = N // RB
EB = 8000
GRID_E = E // EB


def _full(shape):
    return pl.BlockSpec(shape, lambda i: tuple(0 for _ in shape))


def _rows(width):
    return pl.BlockSpec((RB, width), lambda i: (i, 0))


def _dot(a, b):
    return jnp.dot(a, b, preferred_element_type=jnp.float32)


def _tc_a_body(x, degp, ego_W1, ego_b1, ego_W2, ego_b2, g1_W,
               h_ego_o, xw1s_o, dinv_o):
    xb = x[...]
    dp = degp[...]
    h_ego_o[...] = _dot(_elu(_dot(xb, ego_W1[...]) + ego_b1[...]),
                        ego_W2[...]) + ego_b2[...]
    deg = 1.0 + dp[:, 0:1] + dp[:, 1:2]
    dinv = lax.rsqrt(deg)
    dinv_o[...] = dinv
    xw1s_o[...] = _dot(xb, g1_W[...]) * dinv


def _tc_a(x, degp, ego_W1, ego_b1, ego_W2, ego_b2, g1_W):
    return pl.pallas_call(
        _tc_a_body,
        grid=(GRID_N,),
        in_specs=[
            _rows(IN_DIM), _rows(2),
            _full((IN_DIM, H)), _full((H,)), _full((H, H)), _full((H,)),
            _full((IN_DIM, H)),
        ],
        out_specs=[_rows(H), _rows(H), _rows(1)],
        out_shape=[
            jax.ShapeDtypeStruct((N, H), jnp.float32),
            jax.ShapeDtypeStruct((N, H), jnp.float32),
            jax.ShapeDtypeStruct((N, 1), jnp.float32),
        ],
    )(x, degp, ego_W1, ego_b1, ego_W2, ego_b2, g1_W)


def _tc_b_body(a0, a1, xws, dinv, gb, lng, lnb, W2, xw2s_o):
    dv = dinv[...]
    g1out = dv * (a0[...] + a1[...] + xws[...]) + gb[...]
    h1 = _elu(_ln(g1out, lng[...], lnb[...]))
    xw2s_o[...] = _dot(h1, W2[...]) * dv


def _tc_b(a0, a1, xws, dinv, gb, lng, lnb, W2):
    return pl.pallas_call(
        _tc_b_body,
        grid=(GRID_N,),
        in_specs=[
            _rows(H), _rows(H), _rows(H), _rows(1),
            _full((H,)), _full((H,)), _full((H,)), _full((H, H)),
        ],
        out_specs=[_rows(H)],
        out_shape=[jax.ShapeDtypeStruct((N, H), jnp.float32)],
    )(a0, a1, xws, dinv, gb, lng, lnb, W2)


def _tc_c_body(a0, a1, xws, dinv, gb, lng, lnb, h_o):
    g2out = dinv[...] * (a0[...] + a1[...] + xws[...]) + gb[...]
    h_o[...] = _elu(_ln(g2out, lng[...], lnb[...]))


def _tc_c(a0, a1, xws, dinv, gb, lng, lnb):
    return pl.pallas_call(
        _tc_c_body,
        grid=(GRID_N,),
        in_specs=[
            _rows(H), _rows(H), _rows(H), _rows(1),
            _full((H,)), _full((H,)), _full((H,)),
        ],
        out_specs=[_rows(H)],
        out_shape=[jax.ShapeDtypeStruct((N, H), jnp.float32)],
    )(a0, a1, xws, dinv, gb, lng, lnb)


def _tc_d_body(hs, hd, eW1, eb1, eW2, eb2, p_o):
    hsb = hs[...]
    hdb = hd[...]
    w1 = eW1[...]
    d = hdb - hsb
    sim = jnp.exp(-jnp.sum(d * d, axis=-1, keepdims=True) / (2.0 * H))
    z = (_dot(hdb, w1[0:H]) + _dot(hsb, w1[H:2 * H])
         + sim * w1[2 * H:2 * H + 1] + eb1[...])
    u = _elu(z)
    t = jnp.sum(u * eW2[...][:, 0], axis=-1, keepdims=True) + eb2[...]
    wgt = jax.nn.sigmoid(t)
    p_o[...] = jnp.concatenate(
        [hsb * wgt, wgt, jnp.zeros((hsb.shape[0], PW - H - 1), jnp.float32)],
        axis=-1)


def _tc_d(hs, hd, eW1, eb1, eW2, eb2):
    return pl.pallas_call(
        _tc_d_body,
        grid=(GRID_E,),
        in_specs=[
            pl.BlockSpec((EB, H), lambda i: (i, 0)),
            pl.BlockSpec((EB, H), lambda i: (i, 0)),
            _full((2 * H + 1, 16)), _full((16,)), _full((16, 1)), _full((1,)),
        ],
        out_specs=[pl.BlockSpec((EB, PW), lambda i: (i, 0))],
        out_shape=[jax.ShapeDtypeStruct((E, PW), jnp.float32)],
    )(hs, hd, eW1, eb1, eW2, eb2)


def _tc_e_body(h_ego, h, ae0, ae1,
               out_W1, out_b1, out_W2, out_b2, out_W3, out_b3,
               loc_W1, loc_b1, loc_W2, loc_b2,
               mu_W1, mu_b1, mu_W2, mu_b2,
               lv_W1, lv_b1, lv_W2, lv_b2,
               yf_o, yl_o, mu_o, lv_o):
    he = h_ego[...]
    hb = h[...]
    a = ae0[...] + ae1[...]
    h_exp = a[:, 0:H] / jnp.maximum(a[:, H:H + 1], 1e-8)
    h_full = jnp.concatenate([he, hb, h_exp], axis=-1)

    def softmax(v):
        m = jnp.max(v, axis=-1, keepdims=True)
        e = jnp.exp(v - m)
        return e / jnp.sum(e, axis=-1, keepdims=True)

    o = _elu(_dot(h_full, out_W1[...]) + out_b1[...])
    o = _elu(_dot(o, out_W2[...]) + out_b2[...])
    yf_o[...] = softmax(_dot(o, out_W3[...]) + out_b3[...])
    yl_o[...] = softmax(_dot(_elu(_dot(he, loc_W1[...]) + loc_b1[...]),
                             loc_W2[...]) + loc_b2[...])
    mu_o[...] = _dot(_elu(_dot(h_full, mu_W1[...]) + mu_b1[...]),
                     mu_W2[...]) + mu_b2[...]
    lv_o[...] = jnp.clip(_dot(_elu(_dot(h_full, lv_W1[...]) + lv_b1[...]),
                              lv_W2[...]) + lv_b2[...], -5.0, 5.0)


def _tc_e(h_ego, h, ae0, ae1, *weights):
    wspecs = [_full(w.shape) for w in weights]
    return pl.pallas_call(
        _tc_e_body,
        grid=(GRID_N,),
        in_specs=[_rows(H), _rows(H), _rows(PW), _rows(PW)] + wspecs,
        out_specs=[_rows(O), _rows(O), _rows(T), _rows(T)],
        out_shape=[
            jax.ShapeDtypeStruct((N, O), jnp.float32),
            jax.ShapeDtypeStruct((N, O), jnp.float32),
            jax.ShapeDtypeStruct((N, T), jnp.float32),
            jax.ShapeDtypeStruct((N, T), jnp.float32),
        ],
    )(h_ego, h, ae0, ae1, *weights)


def kernel(x, edge_index, ego_W1, ego_b1, ego_W2, ego_b2, g1_W, g1_b,
           g2_W, g2_b, ln1_g, ln1_b, ln2_g, ln2_b, exp_W1, exp_b1,
           exp_W2, exp_b2, out_W1, out_b1, out_W2, out_b2, out_W3, out_b3,
           loc_W1, loc_b1, loc_W2, loc_b2, mu_W1, mu_b1, mu_W2, mu_b2,
           lv_W1, lv_b1, lv_W2, lv_b2):
    src = edge_index[0]
    dst = edge_index[1]

    deg_parts = _sc_deg(dst)
    degp = jnp.stack([deg_parts[0, :N, 0], deg_parts[1, :N, 0]], axis=-1)

    h_ego, xw1s, dinv = _tc_a(x, degp, ego_W1, ego_b1, ego_W2, ego_b2, g1_W)

    a1 = _sc_gcn_edges(xw1s, src, dst)
    (xw2s,) = _tc_b(a1[0, :N], a1[1, :N], xw1s, dinv,
                    g1_b, ln1_g, ln1_b, g2_W)

    a2 = _sc_gcn_edges(xw2s, src, dst)
    (h,) = _tc_c(a2[0, :N], a2[1, :N], xw2s, dinv, g2_b, ln2_g, ln2_b)

    hs, hd = _sc_gather_pairs(h, src, dst)
    (p,) = _tc_d(hs, hd, exp_W1, exp_b1, exp_W2, exp_b2)

    ae = _sc_scatter_msg(p, dst)
    yf, yl, mu, lv = _tc_e(
        h_ego, h, ae[0, :N], ae[1, :N],
        out_W1, out_b1, out_W2, out_b2, out_W3, out_b3,
        loc_W1, loc_b1, loc_W2, loc_b2,
        mu_W1, mu_b1, mu_W2, mu_b2,
        lv_W1, lv_b1, lv_W2, lv_b2)
    return (yf, yl, mu, lv)
